# baseline (speedup 1.0000x reference)
import jax
import jax.numpy as jnp
from jax import lax
from jax.experimental import pallas as pl
from jax.experimental.pallas import tpu as pltpu, tpu_sc as plsc

VOCAB = 100000
EMBED_DIM = 64
BATCH = 4096
SEQ = 50

NC = 2
NS = 16
NW = NC * NS
LANES = 16

VPAD = 100096
NVB = VPAD // 128
NVB_FULL = VOCAB // 128
TAIL_W = VOCAB - NVB_FULL * 128
BLOCKS_PER_W = -(-NVB // NW)

BW = BATCH // NW


def _fmt_kernel(table_t, t_tail, s_out, tin, tbuf, isems, osems):
    w = lax.axis_index("s") * NC + lax.axis_index("c")

    def fire_in(i, b):
        vb = w + i * NW

        @pl.when(vb < NVB_FULL)
        def _full():
            pltpu.async_copy(
                table_t.at[:, pl.ds(vb * 128, 128)], tin.at[b], isems.at[b]
            )

        @pl.when(vb == NVB_FULL)
        def _tail():
            pltpu.async_copy(t_tail, tin.at[b], isems.at[b])

    def wait_in(i, b):
        vb = w + i * NW

        @pl.when(vb <= NVB_FULL)
        def _w():
            pltpu.make_async_copy(
                table_t.at[:, pl.ds(0, 128)], tin.at[b], isems.at[b]
            ).wait()

    def fire_out(i, b):
        vb = w + i * NW

        @pl.when(vb <= NVB_FULL)
        def _w():
            pltpu.async_copy(
                tbuf.at[b, :, pl.ds(0, 128)],
                s_out.at[pl.ds(vb * 128, 128)],
                osems.at[b],
            )

    def wait_out(i, b):
        vb = w + i * NW

        @pl.when(vb <= NVB_FULL)
        def _w():
            pltpu.make_async_copy(
                s_out.at[pl.ds(0, 128)],
                tbuf.at[b, :, pl.ds(0, 128)],
                osems.at[b],
            ).wait()

    lane_iota = lax.iota(jnp.int32, LANES)
    tok_ids = [t * LANES + lane_iota for t in range(128 // LANES)]

    def transpose_block(b):
        def per_d(d, carry):
            splat_d = jnp.full((LANES,), 0, jnp.int32) + d
            for t in range(128 // LANES):
                vals = tin[b, d, pl.ds(t * LANES, LANES)]
                plsc.store_scatter(tbuf.at[b], [tok_ids[t], splat_d], vals)
            return carry

        lax.fori_loop(0, EMBED_DIM, per_d, 0, unroll=8)

    for b in range(2):
        fire_in(b, b)

    NOUT_A = (BLOCKS_PER_W + 1) // 2

    def body(o, carry):
        for b in range(2):
            i = o * 2 + b

            @pl.when(i < BLOCKS_PER_W)
            def _do(i=i, b=b):
                wait_in(i, b)

                @pl.when(i >= 2)
                def _wprev():
                    wait_out(i - 2, b)

                transpose_block(b)
                fire_out(i, b)

                @pl.when(i + 2 < BLOCKS_PER_W)
                def _nxt():
                    fire_in(i + 2, b)

        return carry

    lax.fori_loop(0, NOUT_A, body, 0)

    if BLOCKS_PER_W >= 2:
        wait_out(BLOCKS_PER_W - 2, (BLOCKS_PER_W - 2) % 2)
    wait_out(BLOCKS_PER_W - 1, (BLOCKS_PER_W - 1) % 2)


def _lookup_kernel(s_tab, x_t, out_p, idx_v, gbuf, tbuf, gsems, osems):
    w = lax.axis_index("s") * NC + lax.axis_index("c")
    col0 = w * BW

    pltpu.sync_copy(x_t.at[:, pl.ds(col0, BW)], idx_v)

    def fire_gather(s, b):
        pltpu.async_copy(s_tab.at[idx_v.at[s]], gbuf.at[b], gsems.at[b])

    def wait_gather(b):
        pltpu.make_async_copy(
            s_tab.at[pl.ds(0, BW)], gbuf.at[b], gsems.at[b]
        ).wait()

    def fire_out(s, b):
        pltpu.async_copy(
            tbuf.at[b, :, pl.ds(0, BW)],
            out_p.at[s, :, pl.ds(col0, BW)],
            osems.at[b],
        )

    def wait_out(b):
        pltpu.make_async_copy(
            s_tab.at[pl.ds(0, EMBED_DIM), pl.ds(0, BW)],
            tbuf.at[b, :, pl.ds(0, BW)],
            osems.at[b],
        ).wait()

    zeros16 = jnp.zeros((LANES,), jnp.float32)
    lane_iota = lax.iota(jnp.int32, LANES)

    def mask_pass(s, b):
        for g in range(BW // LANES):
            vec = idx_v[s, pl.ds(g * LANES, LANES)]

            @pl.when(jnp.min(vec) == 0)
            def _zero(vec=vec, g=g, b=b):
                msk = vec == 0
                rid = g * LANES + lane_iota

                def zcol(col, carry):
                    plsc.store_scatter(
                        gbuf.at[b],
                        [rid, jnp.full((LANES,), 0, jnp.int32) + col],
                        zeros16,
                        mask=msk,
                    )
                    return carry

                lax.fori_loop(0, EMBED_DIM, zcol, 0)

    d_ids = [dk * LANES + lane_iota for dk in range(EMBED_DIM // LANES)]

    def transpose_block(b):
        def per_tok(tok, carry):
            splat_tok = jnp.full((LANES,), 0, jnp.int32) + tok
            for dk in range(EMBED_DIM // LANES):
                vals = gbuf[b, tok, pl.ds(dk * LANES, LANES)]
                plsc.store_scatter(tbuf.at[b], [d_ids[dk], splat_tok], vals)
            return carry

        lax.fori_loop(0, BW, per_tok, 0, unroll=8)

    NB = 2
    for b in range(NB):
        fire_gather(b, b)

    def body(outer, carry):
        for b in range(NB):
            s = outer * NB + b
            wait_gather(b)
            mask_pass(s, b)

            @pl.when(outer >= 1)
            def _wprev():
                wait_out(b)

            transpose_block(b)
            fire_out(s, b)

            @pl.when(outer < SEQ // NB - 1)
            def _nxt():
                fire_gather(s + NB, b)

        return carry

    lax.fori_loop(0, SEQ // NB, body, 0)

    for b in range(NB):
        wait_out(b)


_CPARAMS = pltpu.CompilerParams(
    use_tc_tiling_on_sc=True, needs_layout_passes=False
)
_MESH = dict(core_axis_name="c", subcore_axis_name="s")


@jax.jit
def kernel(x, table):
    table_t = table.T
    x_t = x.T

    fmt = pl.kernel(
        _fmt_kernel,
        out_type=jax.ShapeDtypeStruct((VPAD, 128), jnp.float32),
        mesh=plsc.VectorSubcoreMesh(**_MESH),
        compiler_params=_CPARAMS,
        scratch_types=[
            pltpu.VMEM((2, EMBED_DIM, 128), jnp.float32),
            pltpu.VMEM((2, 128, 129), jnp.float32),
            pltpu.SemaphoreType.DMA((2,)),
            pltpu.SemaphoreType.DMA((2,)),
        ],
    )
    t_tail = jnp.pad(
        lax.slice(table_t, (0, NVB_FULL * 128), (EMBED_DIM, VOCAB)),
        ((0, 0), (0, 128 - TAIL_W)),
    )
    s_tab = fmt(table_t, t_tail)

    lookup = pl.kernel(
        _lookup_kernel,
        out_type=jax.ShapeDtypeStruct((SEQ, EMBED_DIM, BATCH), jnp.float32),
        mesh=plsc.VectorSubcoreMesh(**_MESH),
        compiler_params=_CPARAMS,
        scratch_types=[
            pltpu.VMEM((SEQ, BW), jnp.int32),
            pltpu.VMEM((2, BW, 128), jnp.float32),
            pltpu.VMEM((2, EMBED_DIM, BW + 1), jnp.float32),
            pltpu.SemaphoreType.DMA((2,)),
            pltpu.SemaphoreType.DMA((2,)),
        ],
    )
    out_p = lookup(s_tab, x_t)

    return jnp.transpose(out_p, (2, 0, 1))


# speedup vs baseline: 1.6308x; 1.6308x over previous
import jax
import jax.numpy as jnp
from jax import lax
from jax.experimental import pallas as pl
from jax.experimental.pallas import tpu as pltpu, tpu_sc as plsc

VOCAB = 100000
EMBED_DIM = 64
BATCH = 4096
SEQ = 50

NC = 2
NS = 16
NW = NC * NS
LANES = 16

ROWS = BATCH * SEQ
CHUNK = 128
ROWS_PER_W = ROWS // NW
NCHUNK = ROWS_PER_W // CHUNK
G = 5
GROUP = G * CHUNK
NG = NCHUNK // G
NBUF = 2
NOUTER = NG // NBUF

assert ROWS % (NW * CHUNK) == 0 and NCHUNK % G == 0 and NG % NBUF == 0


def _emb_kernel(table_hbm, idx_hbm, out_hbm, idx_v, rows_v, gsems, osems):
    wid = lax.axis_index("s") * NC + lax.axis_index("c")
    row_base = wid * ROWS_PER_W

    pltpu.sync_copy(idx_hbm.at[wid], idx_v)

    def fire_gathers(g, b):
        for j in range(G):
            pltpu.async_copy(
                table_hbm.at[idx_v.at[g * G + j]],
                rows_v.at[b, pl.ds(j * CHUNK, CHUNK)],
                gsems.at[b],
            )

    def drain_gathers(b):
        for _ in range(G):
            pltpu.make_async_copy(
                table_hbm.at[pl.ds(0, CHUNK)],
                rows_v.at[b, pl.ds(0, CHUNK)],
                gsems.at[b],
            ).wait()

    def fire_out(g, b):
        pltpu.async_copy(
            rows_v.at[b], out_hbm.at[pl.ds(row_base + g * GROUP, GROUP)],
            osems.at[b],
        )

    def wait_out(b):
        pltpu.make_async_copy(
            table_hbm.at[pl.ds(0, GROUP)], rows_v.at[b], osems.at[b]
        ).wait()

    zeros16 = jnp.zeros((LANES,), jnp.float32)
    lane_iota = lax.iota(jnp.int32, LANES)

    def mask_pass(g, b):
        for gg in range(GROUP // LANES):
            vec = idx_v[g * G + gg // (CHUNK // LANES),
                        pl.ds((gg % (CHUNK // LANES)) * LANES, LANES)]

            @pl.when(jnp.min(vec) == 0)
            def _zero(vec=vec, gg=gg, b=b):
                msk = vec == 0
                rid = gg * LANES + lane_iota

                def zcol(col, carry):
                    plsc.store_scatter(
                        rows_v.at[b],
                        [rid, jnp.full((LANES,), 0, jnp.int32) + col],
                        zeros16,
                        mask=msk,
                    )
                    return carry

                lax.fori_loop(0, EMBED_DIM, zcol, 0)

    for b in range(NBUF):
        fire_gathers(b, b)

    def body(outer, carry):
        for b in range(NBUF):
            g = outer * NBUF + b
            drain_gathers(b)
            mask_pass(g, b)
            fire_out(g, b)

            @pl.when(outer < NOUTER - 1)
            def _next(g=g, b=b):
                wait_out(b)
                fire_gathers(g + NBUF, b)

        return carry

    lax.fori_loop(0, NOUTER, body, 0)

    for b in range(NBUF):
        wait_out(b)


@jax.jit
def kernel(x, table):
    idx3d = jnp.reshape(x.astype(jnp.int32), (NW, NCHUNK, CHUNK))
    run = pl.kernel(
        _emb_kernel,
        out_type=jax.ShapeDtypeStruct((ROWS, EMBED_DIM), jnp.float32),
        mesh=plsc.VectorSubcoreMesh(core_axis_name="c", subcore_axis_name="s"),
        compiler_params=pltpu.CompilerParams(
            use_tc_tiling_on_sc=False,
            needs_layout_passes=False,
            skip_device_barrier=True,
            disable_bounds_checks=True,
            disable_semaphore_checks=True,
        ),
        scratch_types=[
            pltpu.VMEM((NCHUNK, CHUNK), jnp.int32),
            pltpu.VMEM((NBUF, GROUP, EMBED_DIM), jnp.float32),
            pltpu.SemaphoreType.DMA((NBUF,)),
            pltpu.SemaphoreType.DMA((NBUF,)),
        ],
    )
    out = run(table, idx3d)
    return out.reshape(BATCH, SEQ, EMBED_DIM)
